# baseline (speedup 1.0000x reference)
import functools

import numpy as np
import jax
import jax.numpy as jnp
from jax import lax
from jax.experimental import pallas as pl
from jax.experimental.pallas import tpu as pltpu
from jax.experimental.pallas import tpu_sc as plsc

_SIZE_IMG = 512
_STRIDE = 32
_N_ANCHOR = 9
_EPS = 1e-4
_B = 64
_N_GT = 128


def _anchor_constants():
    hf = _SIZE_IMG // _STRIDE
    wf = _SIZE_IMG // _STRIDE
    smax = 2 ** _SIZE_IMG.bit_length()
    scales = np.array([smax >> 3, smax >> 2, smax >> 1], dtype=np.float32)
    sqrt2 = 1.4142135624
    ratios = np.array([[sqrt2, sqrt2 / 2.0], [1.0, 1.0], [sqrt2 / 2.0, sqrt2]],
                      dtype=np.float32)
    hw_one = np.concatenate([np.outer(scales, ratios[i]) for i in range(3)], axis=0)
    vy = np.arange(hf, dtype=np.float32)
    vx = np.arange(wf, dtype=np.float32)
    yy, xx = np.meshgrid(vy, vx, indexing='ij')
    coords = np.stack([yy, xx], axis=-1)[:, :, None, :] * _STRIDE + _STRIDE // 2
    coords = np.tile(coords, (1, 1, _N_ANCHOR, 1))
    hw = np.tile(hw_one[None, None, :, :], (hf, wf, 1, 1))
    ac_abs = np.concatenate([coords - 0.5 * hw, coords + 0.5 * hw], axis=-1).reshape(-1, 4)
    ac = (ac_abs / float(_SIZE_IMG)).astype(np.float32)
    mask = ((ac[:, 0] >= -0.2) & (ac[:, 1] >= -0.2)
            & (ac[:, 2] <= 1.2) & (ac[:, 3] <= 1.2)
            & (ac[:, 2] > ac[:, 0]) & (ac[:, 3] > ac[:, 1]))
    ac = ac[mask]
    h_r = np.maximum(ac[:, 2] - ac[:, 0], np.float32(_EPS))
    w_r = np.maximum(ac[:, 3] - ac[:, 1], np.float32(_EPS))
    yctr = ac[:, 0] + np.float32(0.5) * h_r
    xctr = ac[:, 1] + np.float32(0.5) * w_r
    area = (ac[:, 2] - ac[:, 0]) * (ac[:, 3] - ac[:, 1])
    iou_rows = np.stack([ac[:, 0], ac[:, 1], ac[:, 2], ac[:, 3], area],
                        axis=0).astype(np.float32)
    delta_consts = np.stack([h_r, w_r, yctr, xctr], axis=0).astype(np.float32)
    return iou_rows, delta_consts


_AC_CONST, _DELTA_CONST = _anchor_constants()
_N_AC = _AC_CONST.shape[1]
_REV_CONST = np.broadcast_to(
    (127.0 - np.arange(_N_GT, dtype=np.float32))[:, None], (_N_GT, _N_AC)).copy()

_BB = 8


def _tc_body(gt_ref, ac_ref, rev_ref, bgg_ref, label_ref):
    for k in range(_BB):
        _tc_one(k, gt_ref, ac_ref, rev_ref, bgg_ref, label_ref)


def _tc_one(k, gt_ref, ac_ref, rev_ref, bgg_ref, label_ref):
    gy0 = gt_ref[k, :, 0:1]
    gx0 = gt_ref[k, :, 1:2]
    gy1 = gt_ref[k, :, 2:3]
    gx1 = gt_ref[k, :, 3:4]
    ay0 = ac_ref[0:1, :]
    ax0 = ac_ref[1:2, :]
    ay1 = ac_ref[2:3, :]
    ax1 = ac_ref[3:4, :]
    area_ac = ac_ref[4:5, :]

    iy = jnp.maximum(jnp.minimum(ay1, gy1) - jnp.maximum(ay0, gy0), 0.0)
    ix = jnp.maximum(jnp.minimum(ax1, gx1) - jnp.maximum(ax0, gx0), 0.0)
    inter = iy * ix
    area_gt = (gy1 - gy0) * (gx1 - gx0)
    union = area_ac + area_gt - inter
    iou = inter / (union + _EPS)
    s = jnp.round(iou * 10000.0)

    enc = s * 128.0 + rev_ref[...]
    enc_max = jnp.max(enc, axis=0, keepdims=True)

    gmax = jnp.max(s, axis=1, keepdims=True)
    gmax2 = jnp.where(gmax > 100.0, gmax, -1.0)
    isb = s == gmax2
    posx = jnp.any(isb, axis=0, keepdims=True)
    pos = (enc_max >= 640000.0) | posx
    neg = (enc_max < 384000.0) & jnp.logical_not(pos)
    label_ref[k] = jnp.where(pos, 1.0, jnp.where(neg, 0.0, -1.0))

    ei = enc_max.astype(jnp.int32)
    b = pl.program_id(0) * _BB + k
    bgg_ref[k] = (127 - (ei & 127)) + b * 128


def _tc_call(bx_gt, ac_const, rev_const, interpret=False):
    return pl.pallas_call(
        _tc_body,
        grid=(_B // _BB,),
        in_specs=[
            pl.BlockSpec((_BB, _N_GT, 4), lambda b: (b, 0, 0)),
            pl.BlockSpec((5, _N_AC), lambda b: (0, 0)),
            pl.BlockSpec((_N_GT, _N_AC), lambda b: (0, 0)),
        ],
        out_specs=[
            pl.BlockSpec((_BB, 1, _N_AC), lambda b: (b, 0, 0)),
            pl.BlockSpec((_BB, 1, _N_AC), lambda b: (b, 0, 0)),
        ],
        out_shape=[
            jax.ShapeDtypeStruct((_B, 1, _N_AC), jnp.int32),
            jax.ShapeDtypeStruct((_B, 1, _N_AC), jnp.float32),
        ],
        compiler_params=pltpu.CompilerParams(
            dimension_semantics=("arbitrary",),
        ),
        interpret=interpret,
    )(bx_gt, ac_const, rev_const)


_N_FLAT = _B * _N_AC
_N_TAB = _B * _N_GT
_LN2 = float(np.log(2.0))


def _sc_log(x):
    bits = plsc.bitcast(x, jnp.int32)
    e = (bits >> 23) - 127
    m = plsc.bitcast((bits & 0x007FFFFF) | 0x3F800000, jnp.float32)
    big = m > 1.4142135623730951
    m = jnp.where(big, m * 0.5, m)
    ef = jnp.where(big, e + 1, e).astype(jnp.float32)
    z = (m - 1.0) / (m + 1.0)
    z2 = z * z
    p = 2.0 + z2 * (2.0 / 3.0 + z2 * (2.0 / 5.0 + z2 * (2.0 / 7.0
                    + z2 * (2.0 / 9.0 + z2 * (2.0 / 11.0)))))
    return ef * _LN2 + z * p


def _sc_delta_kernel(gt_hbm, idx_hbm,
                     hr_hbm, wr_hbm, ycr_hbm, xcr_hbm,
                     ty_hbm, tx_hbm, th_hbm, tw_hbm,
                     gt_v, idx_v, hr_v, wr_v, ycr_v, xcr_v,
                     ty_v, tx_v, th_v, tw_v):
    nw = 32
    chunk = _N_FLAT // nw
    wid = lax.axis_index("s") * 2 + lax.axis_index("c")
    base = wid * chunk
    pltpu.sync_copy(gt_hbm, gt_v)
    pltpu.sync_copy(idx_hbm.at[pl.ds(base, chunk)], idx_v)
    pltpu.sync_copy(hr_hbm.at[pl.ds(base, chunk)], hr_v)
    pltpu.sync_copy(wr_hbm.at[pl.ds(base, chunk)], wr_v)
    pltpu.sync_copy(ycr_hbm.at[pl.ds(base, chunk)], ycr_v)
    pltpu.sync_copy(xcr_hbm.at[pl.ds(base, chunk)], xcr_v)

    def body(g, carry):
        o = g * 16
        fi = idx_v[pl.ds(o, 16)] * 4
        my0 = plsc.load_gather(gt_v, [fi])
        mx0 = plsc.load_gather(gt_v, [fi + 1])
        my1 = plsc.load_gather(gt_v, [fi + 2])
        mx1 = plsc.load_gather(gt_v, [fi + 3])
        h_r = hr_v[pl.ds(o, 16)]
        w_r = wr_v[pl.ds(o, 16)]
        yctr_r = ycr_v[pl.ds(o, 16)]
        xctr_r = xcr_v[pl.ds(o, 16)]
        h_l = my1 - my0
        w_l = mx1 - mx0
        yctr_l = my0 + 0.5 * h_l
        xctr_l = mx0 + 0.5 * w_l
        ty_v[pl.ds(o, 16)] = (yctr_l - yctr_r) / h_r
        tx_v[pl.ds(o, 16)] = (xctr_l - xctr_r) / w_r
        th_v[pl.ds(o, 16)] = _sc_log(jnp.maximum(h_l, _EPS) / h_r)
        tw_v[pl.ds(o, 16)] = _sc_log(jnp.maximum(w_l, _EPS) / w_r)
        return carry

    lax.fori_loop(0, chunk // 16, body, 0, unroll=4)
    pltpu.sync_copy(ty_v, ty_hbm.at[pl.ds(base, chunk)])
    pltpu.sync_copy(tx_v, tx_hbm.at[pl.ds(base, chunk)])
    pltpu.sync_copy(th_v, th_hbm.at[pl.ds(base, chunk)])
    pltpu.sync_copy(tw_v, tw_hbm.at[pl.ds(base, chunk)])


def _sc_call(gt_flat, idx, hr, wr, ycr, xcr):
    chunk = _N_FLAT // 32
    mesh = plsc.VectorSubcoreMesh(core_axis_name="c", subcore_axis_name="s")
    f = functools.partial(
        pl.kernel,
        mesh=mesh,
        compiler_params=pltpu.CompilerParams(needs_layout_passes=False),
        out_type=[jax.ShapeDtypeStruct((_N_FLAT,), jnp.float32)] * 4,
        scratch_types=[
            pltpu.VMEM((_N_TAB * 4,), jnp.float32),
            pltpu.VMEM((chunk,), jnp.int32),
            pltpu.VMEM((chunk,), jnp.float32),
            pltpu.VMEM((chunk,), jnp.float32),
            pltpu.VMEM((chunk,), jnp.float32),
            pltpu.VMEM((chunk,), jnp.float32),
            pltpu.VMEM((chunk,), jnp.float32),
            pltpu.VMEM((chunk,), jnp.float32),
            pltpu.VMEM((chunk,), jnp.float32),
            pltpu.VMEM((chunk,), jnp.float32),
        ],
    )(_sc_delta_kernel)
    return f(gt_flat, idx, hr, wr, ycr, xcr)


_CST_FLAT = np.tile(_DELTA_CONST[:, None, :], (1, _B, 1)).reshape(4, _N_FLAT)


def kernel(bx_gt):
    ac_const = jnp.asarray(_AC_CONST)
    rev_const = jnp.asarray(_REV_CONST)
    bgg, label = _tc_call(bx_gt, ac_const, rev_const)

    gt_flat = bx_gt.reshape(_N_TAB * 4)
    idx = bgg.reshape(_N_FLAT)
    hr = jnp.asarray(_CST_FLAT[0])
    wr = jnp.asarray(_CST_FLAT[1])
    ycr = jnp.asarray(_CST_FLAT[2])
    xcr = jnp.asarray(_CST_FLAT[3])
    ty, tx, th, tw = _sc_call(gt_flat, idx, hr, wr, ycr, xcr)

    delta = jnp.stack([tx, ty, tw, th], axis=-1).reshape(_B, _N_AC, 4)
    return delta, label.reshape(_B, _N_AC)


# speedup vs baseline: 1.0105x; 1.0105x over previous
import functools

import numpy as np
import jax
import jax.numpy as jnp
from jax import lax
from jax.experimental import pallas as pl
from jax.experimental.pallas import tpu as pltpu
from jax.experimental.pallas import tpu_sc as plsc

_SIZE_IMG = 512
_STRIDE = 32
_N_ANCHOR = 9
_EPS = 1e-4
_B = 64
_N_GT = 128


def _anchor_constants():
    hf = _SIZE_IMG // _STRIDE
    wf = _SIZE_IMG // _STRIDE
    smax = 2 ** _SIZE_IMG.bit_length()
    scales = np.array([smax >> 3, smax >> 2, smax >> 1], dtype=np.float32)
    sqrt2 = 1.4142135624
    ratios = np.array([[sqrt2, sqrt2 / 2.0], [1.0, 1.0], [sqrt2 / 2.0, sqrt2]],
                      dtype=np.float32)
    hw_one = np.concatenate([np.outer(scales, ratios[i]) for i in range(3)], axis=0)
    vy = np.arange(hf, dtype=np.float32)
    vx = np.arange(wf, dtype=np.float32)
    yy, xx = np.meshgrid(vy, vx, indexing='ij')
    coords = np.stack([yy, xx], axis=-1)[:, :, None, :] * _STRIDE + _STRIDE // 2
    coords = np.tile(coords, (1, 1, _N_ANCHOR, 1))
    hw = np.tile(hw_one[None, None, :, :], (hf, wf, 1, 1))
    ac_abs = np.concatenate([coords - 0.5 * hw, coords + 0.5 * hw], axis=-1).reshape(-1, 4)
    ac = (ac_abs / float(_SIZE_IMG)).astype(np.float32)
    mask = ((ac[:, 0] >= -0.2) & (ac[:, 1] >= -0.2)
            & (ac[:, 2] <= 1.2) & (ac[:, 3] <= 1.2)
            & (ac[:, 2] > ac[:, 0]) & (ac[:, 3] > ac[:, 1]))
    ac = ac[mask]
    h_r = np.maximum(ac[:, 2] - ac[:, 0], np.float32(_EPS))
    w_r = np.maximum(ac[:, 3] - ac[:, 1], np.float32(_EPS))
    yctr = ac[:, 0] + np.float32(0.5) * h_r
    xctr = ac[:, 1] + np.float32(0.5) * w_r
    area = (ac[:, 2] - ac[:, 0]) * (ac[:, 3] - ac[:, 1])
    iou_rows = np.stack([ac[:, 0], ac[:, 1], ac[:, 2], ac[:, 3], area],
                        axis=0).astype(np.float32)
    delta_consts = np.stack([h_r, w_r, yctr, xctr], axis=0).astype(np.float32)
    return iou_rows, delta_consts


_AC_CONST, _DELTA_CONST = _anchor_constants()
_N_AC = _AC_CONST.shape[1]
_REV_CONST = np.broadcast_to(
    (127.0 - np.arange(_N_GT, dtype=np.float32))[:, None], (_N_GT, _N_AC)).copy()

_BB = 8


def _tc_body(gt_ref, ac_ref, rev_ref, bgg_ref, label_ref):
    for k in range(_BB):
        _tc_one(k, gt_ref, ac_ref, rev_ref, bgg_ref, label_ref)


def _tc_one(k, gt_ref, ac_ref, rev_ref, bgg_ref, label_ref):
    gy0 = gt_ref[k, :, 0:1]
    gx0 = gt_ref[k, :, 1:2]
    gy1 = gt_ref[k, :, 2:3]
    gx1 = gt_ref[k, :, 3:4]
    ay0 = ac_ref[0:1, :]
    ax0 = ac_ref[1:2, :]
    ay1 = ac_ref[2:3, :]
    ax1 = ac_ref[3:4, :]
    area_ac = ac_ref[4:5, :]

    iy = jnp.maximum(jnp.minimum(ay1, gy1) - jnp.maximum(ay0, gy0), 0.0)
    ix = jnp.maximum(jnp.minimum(ax1, gx1) - jnp.maximum(ax0, gx0), 0.0)
    inter = iy * ix
    area_gt = (gy1 - gy0) * (gx1 - gx0)
    union = area_ac + area_gt - inter
    iou = inter / (union + _EPS)
    s = jnp.round(iou * 10000.0)

    enc = s * 128.0 + rev_ref[...]
    enc_max = jnp.max(enc, axis=0, keepdims=True)

    gmax = jnp.max(s, axis=1, keepdims=True)
    gmax2 = jnp.where(gmax > 100.0, gmax, -1.0)
    isb = s == gmax2
    posx = jnp.any(isb, axis=0, keepdims=True)
    pos = (enc_max >= 640000.0) | posx
    neg = (enc_max < 384000.0) & jnp.logical_not(pos)
    label_ref[k] = jnp.where(pos, 1.0, jnp.where(neg, 0.0, -1.0))

    ei = enc_max.astype(jnp.int32)
    b = pl.program_id(0) * _BB + k
    bgg_ref[k] = (127 - (ei & 127)) + b * 128


def _tc_call(bx_gt, ac_const, rev_const, interpret=False):
    return pl.pallas_call(
        _tc_body,
        grid=(_B // _BB,),
        in_specs=[
            pl.BlockSpec((_BB, _N_GT, 4), lambda b: (b, 0, 0)),
            pl.BlockSpec((5, _N_AC), lambda b: (0, 0)),
            pl.BlockSpec((_N_GT, _N_AC), lambda b: (0, 0)),
        ],
        out_specs=[
            pl.BlockSpec((_BB, 1, _N_AC), lambda b: (b, 0, 0)),
            pl.BlockSpec((_BB, 1, _N_AC), lambda b: (b, 0, 0)),
        ],
        out_shape=[
            jax.ShapeDtypeStruct((_B, 1, _N_AC), jnp.int32),
            jax.ShapeDtypeStruct((_B, 1, _N_AC), jnp.float32),
        ],
        compiler_params=pltpu.CompilerParams(
            dimension_semantics=("arbitrary",),
        ),
        interpret=interpret,
    )(bx_gt, ac_const, rev_const)


_N_FLAT = _B * _N_AC
_N_TAB = _B * _N_GT
_LN2 = float(np.log(2.0))


def _sc_log(x):
    bits = plsc.bitcast(x, jnp.int32)
    e = (bits >> 23) - 127
    m = plsc.bitcast((bits & 0x007FFFFF) | 0x3F800000, jnp.float32)
    big = m > 1.4142135623730951
    m = jnp.where(big, m * 0.5, m)
    ef = jnp.where(big, e + 1, e).astype(jnp.float32)
    z = (m - 1.0) / (m + 1.0)
    z2 = z * z
    p = 2.0 + z2 * (2.0 / 3.0 + z2 * (2.0 / 5.0 + z2 * (2.0 / 7.0
                    + z2 * (2.0 / 9.0 + z2 * (2.0 / 11.0)))))
    return ef * _LN2 + z * p


def _sc_delta_kernel(gt_hbm, idx_hbm,
                     hr_hbm, wr_hbm, ycr_hbm, xcr_hbm,
                     ty_hbm, tx_hbm, th_hbm, tw_hbm,
                     gt_v, idx_v, hr_v, wr_v, ycr_v, xcr_v,
                     ty_v, tx_v, th_v, tw_v):
    nw = 32
    chunk = _N_FLAT // nw
    wid = lax.axis_index("s") * 2 + lax.axis_index("c")
    base = wid * chunk
    pltpu.sync_copy(gt_hbm, gt_v)
    pltpu.sync_copy(idx_hbm.at[pl.ds(base, chunk)], idx_v)
    pltpu.sync_copy(hr_hbm.at[pl.ds(base, chunk)], hr_v)
    pltpu.sync_copy(wr_hbm.at[pl.ds(base, chunk)], wr_v)
    pltpu.sync_copy(ycr_hbm.at[pl.ds(base, chunk)], ycr_v)
    pltpu.sync_copy(xcr_hbm.at[pl.ds(base, chunk)], xcr_v)

    def body(g, carry):
        o = g * 16
        fi = idx_v[pl.ds(o, 16)] * 4
        my0 = plsc.load_gather(gt_v, [fi])
        mx0 = plsc.load_gather(gt_v, [fi + 1])
        my1 = plsc.load_gather(gt_v, [fi + 2])
        mx1 = plsc.load_gather(gt_v, [fi + 3])
        h_r = hr_v[pl.ds(o, 16)]
        w_r = wr_v[pl.ds(o, 16)]
        yctr_r = ycr_v[pl.ds(o, 16)]
        xctr_r = xcr_v[pl.ds(o, 16)]
        h_l = my1 - my0
        w_l = mx1 - mx0
        yctr_l = my0 + 0.5 * h_l
        xctr_l = mx0 + 0.5 * w_l
        ty_v[pl.ds(o, 16)] = (yctr_l - yctr_r) / h_r
        tx_v[pl.ds(o, 16)] = (xctr_l - xctr_r) / w_r
        th_v[pl.ds(o, 16)] = _sc_log(jnp.maximum(h_l, _EPS) / h_r)
        tw_v[pl.ds(o, 16)] = _sc_log(jnp.maximum(w_l, _EPS) / w_r)
        return carry

    lax.fori_loop(0, chunk // 16, body, 0)
    pltpu.sync_copy(ty_v, ty_hbm.at[pl.ds(base, chunk)])
    pltpu.sync_copy(tx_v, tx_hbm.at[pl.ds(base, chunk)])
    pltpu.sync_copy(th_v, th_hbm.at[pl.ds(base, chunk)])
    pltpu.sync_copy(tw_v, tw_hbm.at[pl.ds(base, chunk)])


def _sc_call(gt_flat, idx, hr, wr, ycr, xcr):
    chunk = _N_FLAT // 32
    mesh = plsc.VectorSubcoreMesh(core_axis_name="c", subcore_axis_name="s")
    f = functools.partial(
        pl.kernel,
        mesh=mesh,
        compiler_params=pltpu.CompilerParams(needs_layout_passes=False),
        out_type=[jax.ShapeDtypeStruct((_N_FLAT,), jnp.float32)] * 4,
        scratch_types=[
            pltpu.VMEM((_N_TAB * 4,), jnp.float32),
            pltpu.VMEM((chunk,), jnp.int32),
            pltpu.VMEM((chunk,), jnp.float32),
            pltpu.VMEM((chunk,), jnp.float32),
            pltpu.VMEM((chunk,), jnp.float32),
            pltpu.VMEM((chunk,), jnp.float32),
            pltpu.VMEM((chunk,), jnp.float32),
            pltpu.VMEM((chunk,), jnp.float32),
            pltpu.VMEM((chunk,), jnp.float32),
            pltpu.VMEM((chunk,), jnp.float32),
        ],
    )(_sc_delta_kernel)
    return f(gt_flat, idx, hr, wr, ycr, xcr)


_CST_FLAT = np.tile(_DELTA_CONST[:, None, :], (1, _B, 1)).reshape(4, _N_FLAT)


def kernel(bx_gt):
    ac_const = jnp.asarray(_AC_CONST)
    rev_const = jnp.asarray(_REV_CONST)
    bgg, label = _tc_call(bx_gt, ac_const, rev_const)

    gt_flat = bx_gt.reshape(_N_TAB * 4)
    idx = bgg.reshape(_N_FLAT)
    hr = jnp.asarray(_CST_FLAT[0])
    wr = jnp.asarray(_CST_FLAT[1])
    ycr = jnp.asarray(_CST_FLAT[2])
    xcr = jnp.asarray(_CST_FLAT[3])
    ty, tx, th, tw = _sc_call(gt_flat, idx, hr, wr, ycr, xcr)

    delta = jnp.stack([tx, ty, tw, th], axis=-1).reshape(_B, _N_AC, 4)
    return delta, label.reshape(_B, _N_AC)
